# baseline (speedup 1.0000x reference)
import functools
import math

import jax
import jax.numpy as jnp
from jax import lax
from jax.experimental import pallas as pl
from jax.experimental.pallas import tpu as pltpu
from jax.experimental.pallas import tpu_sc as plsc

_VOCAB = 100000
_DIM = 64
_B = 4096
_L = 200
_BASE_FREQ = 1e-05

_NC = 2
_NS = 16
_NW = _NC * _NS
_LT = _L // 8


def _pe_table():
    def body(o_ref):
        col = lax.broadcasted_iota(jnp.int32, (_L, _DIM), 1).astype(jnp.float32)
        row = lax.broadcasted_iota(jnp.int32, (_L, _DIM), 0).astype(jnp.float32)
        mult = jnp.exp(row * (2.0 * math.log(_BASE_FREQ) / (_L - 1)))
        o_ref[...] = jnp.sin(col * mult)

    return pl.pallas_call(
        body, out_shape=jax.ShapeDtypeStruct((_L, _DIM), jnp.float32)
    )()


def _sc_body(x_hbm, w_hbm, pe_hbm, out_hbm, idx_v, pe_v, rows0, rows1,
             tbuf0, tbuf1, sg0, sg1, ss0, ss1):
    wid = lax.axis_index("s") * _NC + lax.axis_index("c")
    bt = wid

    pltpu.sync_copy(x_hbm.at[:, bt], idx_v)
    pltpu.sync_copy(pe_hbm, pe_v)

    base16 = lax.iota(jnp.int32, 16)
    row_idx = [base16 + 16 * k for k in range(8)]

    def fire_gather(l, rows, sem):
        pltpu.async_copy(w_hbm.at[idx_v.at[l >> 3, l & 7]], rows, sem)

    def wait_gather(rows, sem):
        pltpu.make_async_copy(w_hbm.at[pl.ds(0, 128), :], rows, sem).wait()

    def fire_store(l, tbuf, sem):
        pltpu.async_copy(tbuf, out_hbm.at[l, :, bt], sem)

    def wait_store(tbuf, sem):
        pltpu.make_async_copy(tbuf, out_hbm.at[0, :, 0], sem).wait()

    def transpose_add(l, rows, tbuf):
        lv = lax.broadcast_in_dim(l, (16,), ())

        def ctbody(ct, carry):
            tb = tbuf.at[ct]
            cbase = ct * 8
            for cs in range(8):
                cv = lax.broadcast_in_dim(cbase + cs, (16,), ())
                pev = plsc.load_gather(pe_v, [lv, cv])
                for k in range(8):
                    v = plsc.load_gather(rows, [row_idx[k], cv])
                    tb[cs, pl.ds(16 * k, 16)] = v + pev
            return carry

        lax.fori_loop(0, _DIM // 8, ctbody, 0)

    fire_gather(0, rows0, sg0)

    def outer(t, carry):
        for s, (rows, sg, other_rows, other_sg, tbuf, ss) in enumerate((
                (rows0, sg0, rows1, sg1, tbuf0, ss0),
                (rows1, sg1, rows0, sg0, tbuf1, ss1))):
            l = 2 * t + s

            @pl.when(l + 1 < _L)
            def _():
                fire_gather(l + 1, other_rows, other_sg)

            wait_gather(rows, sg)

            @pl.when(l >= 2)
            def _():
                wait_store(tbuf, ss)

            transpose_add(l, rows, tbuf)
            fire_store(l, tbuf, ss)
        return carry

    lax.fori_loop(0, _L // 2, outer, 0)
    wait_store(tbuf0, ss0)
    wait_store(tbuf1, ss1)


@jax.jit
def kernel(x, W):
    pe = _pe_table()
    x32 = jnp.asarray(x, jnp.int32)
    xr = x32.reshape(_NW, 128, _LT, 8).transpose(2, 0, 3, 1)

    mesh = plsc.VectorSubcoreMesh(core_axis_name="c", subcore_axis_name="s")
    run = pl.kernel(
        _sc_body,
        out_type=jax.ShapeDtypeStruct((_L, _DIM // 8, _NW, 8, 128),
                                      jnp.float32),
        mesh=mesh,
        compiler_params=pltpu.CompilerParams(use_tc_tiling_on_sc=False,
                                             needs_layout_passes=False),
        scratch_types=[
            pltpu.VMEM((_LT, 8, 128), jnp.int32),
            pltpu.VMEM((_L, _DIM), jnp.float32),
            pltpu.VMEM((128, _DIM), jnp.float32),
            pltpu.VMEM((128, _DIM), jnp.float32),
            pltpu.VMEM((_DIM // 8, 8, 128), jnp.float32),
            pltpu.VMEM((_DIM // 8, 8, 128), jnp.float32),
            pltpu.SemaphoreType.DMA,
            pltpu.SemaphoreType.DMA,
            pltpu.SemaphoreType.DMA,
            pltpu.SemaphoreType.DMA,
        ],
    )
    out = run(xr, W, pe)
    return out.transpose(2, 4, 0, 1, 3).reshape(_B, _L, _DIM)


# speedup vs baseline: 1.7065x; 1.7065x over previous
import functools
import math

import jax
import jax.numpy as jnp
from jax import lax
from jax.experimental import pallas as pl
from jax.experimental.pallas import tpu as pltpu
from jax.experimental.pallas import tpu_sc as plsc

_VOCAB = 100000
_DIM = 64
_B = 4096
_L = 200
_BASE_FREQ = 1e-05

_NC = 2
_NS = 16
_NW = _NC * _NS
_LT = _L // 8


def _pe_table():
    def body(o_ref):
        col = lax.broadcasted_iota(jnp.int32, (_L, _DIM), 1).astype(jnp.float32)
        row = lax.broadcasted_iota(jnp.int32, (_L, _DIM), 0).astype(jnp.float32)
        mult = jnp.exp(row * (2.0 * math.log(_BASE_FREQ) / (_L - 1)))
        o_ref[...] = jnp.sin(col * mult)

    return pl.pallas_call(
        body, out_shape=jax.ShapeDtypeStruct((_L, _DIM), jnp.float32)
    )()


def _sc_body(x_hbm, w_hbm, pe_hbm, out_hbm, idx_v, pe_v, rows0, rows1,
             tbuf0, tbuf1, sg0, sg1, ss0, ss1):
    wid = lax.axis_index("s") * _NC + lax.axis_index("c")
    bt = wid

    pltpu.sync_copy(x_hbm.at[:, bt], idx_v)
    pltpu.sync_copy(pe_hbm, pe_v)

    base16 = lax.iota(jnp.int32, 16)
    row_idx = [base16 + 16 * k for k in range(8)]

    def fire_gather(l, rows, sem):
        pltpu.async_copy(w_hbm.at[idx_v.at[l >> 3, l & 7]], rows, sem)

    def wait_gather(rows, sem):
        pltpu.make_async_copy(w_hbm.at[pl.ds(0, 128), :], rows, sem).wait()

    def fire_store(l, tbuf, sem):
        pltpu.async_copy(tbuf, out_hbm.at[l, :, bt], sem)

    def wait_store(tbuf, sem):
        pltpu.make_async_copy(tbuf, out_hbm.at[0, :, 0], sem).wait()

    def transpose_add(l, rows, tbuf):
        lv = lax.broadcast_in_dim(l, (16,), ())

        @plsc.parallel_loop(0, _DIM // 8, unroll=2)
        def ctbody(ct):
            tb = tbuf.at[ct]
            cbase = ct * 8
            for cs in range(8):
                cv = lax.broadcast_in_dim(cbase + cs, (16,), ())
                pev = plsc.load_gather(pe_v, [lv, cv])
                for k in range(8):
                    v = plsc.load_gather(rows, [row_idx[k], cv])
                    tb[cs, pl.ds(16 * k, 16)] = v + pev

    fire_gather(0, rows0, sg0)

    def outer(t, carry):
        for s, (rows, sg, other_rows, other_sg, tbuf, ss) in enumerate((
                (rows0, sg0, rows1, sg1, tbuf0, ss0),
                (rows1, sg1, rows0, sg0, tbuf1, ss1))):
            l = 2 * t + s

            @pl.when(l + 1 < _L)
            def _():
                fire_gather(l + 1, other_rows, other_sg)

            wait_gather(rows, sg)

            @pl.when(l >= 2)
            def _():
                wait_store(tbuf, ss)

            transpose_add(l, rows, tbuf)
            fire_store(l, tbuf, ss)
        return carry

    lax.fori_loop(0, _L // 2, outer, 0)
    wait_store(tbuf0, ss0)
    wait_store(tbuf1, ss1)


@jax.jit
def kernel(x, W):
    pe = _pe_table()
    x32 = jnp.asarray(x, jnp.int32)
    xr = x32.reshape(_NW, 128, _LT, 8).transpose(2, 0, 3, 1)

    mesh = plsc.VectorSubcoreMesh(core_axis_name="c", subcore_axis_name="s")
    run = pl.kernel(
        _sc_body,
        out_type=jax.ShapeDtypeStruct((_L, _DIM // 8, _NW, 8, 128),
                                      jnp.float32),
        mesh=mesh,
        compiler_params=pltpu.CompilerParams(use_tc_tiling_on_sc=False,
                                             needs_layout_passes=False),
        scratch_types=[
            pltpu.VMEM((_LT, 8, 128), jnp.int32),
            pltpu.VMEM((_L, _DIM), jnp.float32),
            pltpu.VMEM((128, _DIM), jnp.float32),
            pltpu.VMEM((128, _DIM), jnp.float32),
            pltpu.VMEM((_DIM // 8, 8, 128), jnp.float32),
            pltpu.VMEM((_DIM // 8, 8, 128), jnp.float32),
            pltpu.SemaphoreType.DMA,
            pltpu.SemaphoreType.DMA,
            pltpu.SemaphoreType.DMA,
            pltpu.SemaphoreType.DMA,
        ],
    )
    out = run(xr, W, pe)
    return out.transpose(2, 4, 0, 1, 3).reshape(_B, _L, _DIM)


# speedup vs baseline: 5.5114x; 3.2297x over previous
import functools
import math

import jax
import jax.numpy as jnp
from jax import lax
from jax.experimental import pallas as pl
from jax.experimental.pallas import tpu as pltpu
from jax.experimental.pallas import tpu_sc as plsc

_VOCAB = 100000
_DIM = 64
_B = 4096
_L = 200
_BASE_FREQ = 1e-05

_NC = 2
_NS = 16
_NW = _NC * _NS
_LT = _L // 8


def _pe_table():
    def body(o_ref):
        col = lax.broadcasted_iota(jnp.int32, (_L, _DIM), 1).astype(jnp.float32)
        row = lax.broadcasted_iota(jnp.int32, (_L, _DIM), 0).astype(jnp.float32)
        mult = jnp.exp(row * (2.0 * math.log(_BASE_FREQ) / (_L - 1)))
        o_ref[...] = jnp.sin(col * mult)

    return pl.pallas_call(
        body, out_shape=jax.ShapeDtypeStruct((_L, _DIM), jnp.float32)
    )()


def _sc_body(x_hbm, w_hbm, pe_hbm, out_hbm, idx_v, pe_v, rows0, rows1,
             tbuf0, tbuf1, sg0, sg1, ss0, ss1):
    wid = lax.axis_index("s") * _NC + lax.axis_index("c")
    bt = wid

    pltpu.sync_copy(x_hbm.at[:, bt], idx_v)
    pltpu.sync_copy(pe_hbm, pe_v)

    base16 = lax.iota(jnp.int32, 16)
    row_idx = [base16 + 16 * k for k in range(8)]

    def fire_gather(l, rows, sem):
        pltpu.async_copy(w_hbm.at[idx_v.at[l >> 3, l & 7]], rows, sem)

    def wait_gather(rows, sem):
        pltpu.make_async_copy(w_hbm.at[pl.ds(0, 128), :], rows, sem).wait()

    def fire_store(l, tbuf, sem):
        pltpu.async_copy(tbuf.at[:, :, pl.ds(0, 128)],
                         out_hbm.at[l, :, bt], sem)

    def wait_store(tbuf, sem):
        pltpu.make_async_copy(tbuf.at[:, :, pl.ds(0, 128)],
                              out_hbm.at[0, :, 0], sem).wait()

    ct_j = [(base16 + 16 * j) >> 3 for j in range(4)]
    cs_j = [(base16 + 16 * j) & 7 for j in range(4)]

    def transpose_add(l, rows, tbuf):
        pes = [pe_v[l, pl.ds(16 * j, 16)] for j in range(4)]

        @plsc.parallel_loop(0, 128, unroll=4)
        def rbody(r):
            rv = lax.broadcast_in_dim(r, (16,), ())
            for j in range(4):
                v = rows[r, pl.ds(16 * j, 16)] + pes[j]
                plsc.store_scatter(tbuf, [ct_j[j], cs_j[j], rv], v)

    fire_gather(0, rows0, sg0)

    def outer(t, carry):
        for s, (rows, sg, other_rows, other_sg, tbuf, ss) in enumerate((
                (rows0, sg0, rows1, sg1, tbuf0, ss0),
                (rows1, sg1, rows0, sg0, tbuf1, ss1))):
            l = 2 * t + s

            @pl.when(l + 1 < _L)
            def _():
                fire_gather(l + 1, other_rows, other_sg)

            wait_gather(rows, sg)

            @pl.when(l >= 2)
            def _():
                wait_store(tbuf, ss)

            transpose_add(l, rows, tbuf)
            fire_store(l, tbuf, ss)
        return carry

    lax.fori_loop(0, _L // 2, outer, 0)
    wait_store(tbuf0, ss0)
    wait_store(tbuf1, ss1)


@jax.jit
def kernel(x, W):
    pe = _pe_table()
    x32 = jnp.asarray(x, jnp.int32)
    xr = x32.reshape(_NW, 128, _LT, 8).transpose(2, 0, 3, 1)

    mesh = plsc.VectorSubcoreMesh(core_axis_name="c", subcore_axis_name="s")
    run = pl.kernel(
        _sc_body,
        out_type=jax.ShapeDtypeStruct((_L, _DIM // 8, _NW, 8, 128),
                                      jnp.float32),
        mesh=mesh,
        compiler_params=pltpu.CompilerParams(use_tc_tiling_on_sc=False,
                                             needs_layout_passes=False),
        scratch_types=[
            pltpu.VMEM((_LT, 8, 128), jnp.int32),
            pltpu.VMEM((_L, _DIM), jnp.float32),
            pltpu.VMEM((128, _DIM), jnp.float32),
            pltpu.VMEM((128, _DIM), jnp.float32),
            pltpu.VMEM((_DIM // 8, 8, 129), jnp.float32),
            pltpu.VMEM((_DIM // 8, 8, 129), jnp.float32),
            pltpu.SemaphoreType.DMA,
            pltpu.SemaphoreType.DMA,
            pltpu.SemaphoreType.DMA,
            pltpu.SemaphoreType.DMA,
        ],
    )
    out = run(xr, W, pe)
    return out.transpose(2, 4, 0, 1, 3).reshape(_B, _L, _DIM)
